# baseline (speedup 1.0000x reference)
import functools

import jax
import jax.numpy as jnp
from jax import lax
from jax.experimental import pallas as pl
from jax.experimental.pallas import tpu as pltpu
from jax.experimental.pallas import tpu_sc as plsc

N = 10000
E = 320000
D = 128

NC = 2
NS = 16
NW = NC * NS
EPW = E // NW
CH = 96
NCHUNK = EPW // CH
TAIL = EPW - NCHUNK * CH
NB = 4
LEFT = NCHUNK % NB
RZ = 80

def _zero_vec16():
    return jnp.zeros((16,), jnp.float32)


@functools.cache
def _deg_kernel():
    return pl.kernel(
        _deg_body,
        out_type=jax.ShapeDtypeStruct((NW * N,), jnp.float32),
        mesh=plsc.VectorSubcoreMesh(core_axis_name="c", subcore_axis_name="s"),
        compiler_params=pltpu.CompilerParams(needs_layout_passes=False),
        scratch_types=[
            pltpu.VMEM((N,), jnp.float32),
            pltpu.VMEM((EPW,), jnp.int32),
        ],
    )


def _deg_body(dst_hbm, out_hbm, hist, dbuf):
    c = lax.axis_index("c")
    s = lax.axis_index("s")
    wid = s * NC + c

    pltpu.sync_copy(dst_hbm.at[pl.ds(wid * EPW, EPW)], dbuf)

    def zero_body(i, carry):
        hist[pl.ds(i * 16, 16)] = _zero_vec16()
        return carry

    lax.fori_loop(0, N // 16, zero_body, 0)

    ones = jnp.ones((16,), jnp.float32)

    def body(j, carry):
        d16 = dbuf[pl.ds(j * 16, 16)]
        plsc.addupdate_scatter(hist, [d16], ones)
        return carry

    lax.fori_loop(0, EPW // 16, body, 0)
    pltpu.sync_copy(hist, out_hbm.at[pl.ds(wid * N, N)])


@functools.cache
def _agg_kernel():
    return pl.kernel(
        _agg_body,
        out_type=jax.ShapeDtypeStruct((NC * N, D), jnp.float32),
        mesh=plsc.VectorSubcoreMesh(core_axis_name="c", subcore_axis_name="s"),
        compiler_params=pltpu.CompilerParams(needs_layout_passes=False),
        scratch_types=(
            [pltpu.VMEM_SHARED((N, D), jnp.float32)]
            + [pltpu.VMEM((CH, D), jnp.float32)] * NB
            + [pltpu.VMEM((CH,), jnp.int32)] * NB
            + [pltpu.VMEM((CH,), jnp.int32)] * NB
            + [pltpu.VMEM((TAIL,), jnp.int32)] * 2
            + [pltpu.SemaphoreType.DMA] * (3 * NB)
        ),
    )


NRCH = N // RZ
RCPT = (NRCH + NS - 1) // NS


def _agg_body(hp_hbm, src_hbm, dst_hbm, out_hbm, acc, *bufs):
    rows = bufs[0:NB]
    sbuf = bufs[NB:2 * NB]
    dbuf = bufs[2 * NB:3 * NB]
    sbuf_t, dbuf_t = bufs[3 * NB:3 * NB + 2]
    gsem = bufs[3 * NB + 2:4 * NB + 2]
    ssem = bufs[4 * NB + 2:5 * NB + 2]
    isem = bufs[5 * NB + 2:6 * NB + 2]
    c = lax.axis_index("c")
    s = lax.axis_index("s")
    wid = s * NC + c
    ebase = wid * EPW

    def zrow(i, carry):
        for j in range(D // 16):
            rows[0][i, pl.ds(j * 16, 16)] = _zero_vec16()
        return carry

    lax.fori_loop(0, RZ, zrow, 0)
    for r in range(RCPT):
        chunk = s + r * NS
        @pl.when(chunk < NRCH)
        def _():
            pltpu.sync_copy(rows[0].at[pl.ds(0, RZ), :], acc.at[pl.ds(chunk * RZ, RZ), :])
    plsc.subcore_barrier()

    def _start_idx(k, b):
        pltpu.async_copy(src_hbm.at[pl.ds(ebase + k * CH, CH)], sbuf[b], isem[b])
        pltpu.async_copy(dst_hbm.at[pl.ds(ebase + k * CH, CH)], dbuf[b], isem[b])

    def _wait_idx(k, b):
        pltpu.make_async_copy(src_hbm.at[pl.ds(ebase + k * CH, CH)], sbuf[b], isem[b]).wait()
        pltpu.make_async_copy(dst_hbm.at[pl.ds(ebase + k * CH, CH)], dbuf[b], isem[b]).wait()

    def _start_gather(b):
        pltpu.async_copy(hp_hbm.at[sbuf[b]], rows[b], gsem[b])

    def _wait_gather(b):
        pltpu.make_async_copy(hp_hbm.at[sbuf[b]], rows[b], gsem[b]).wait()

    def _start_scatter(b):
        pltpu.async_copy(rows[b], acc.at[dbuf[b]], ssem[b], add=True)

    def _wait_scatter(b):
        pltpu.make_async_copy(rows[b], acc.at[dbuf[b]], ssem[b]).wait()

    for b in range(NB):
        _start_idx(b, b)
        _wait_idx(b, b)
        _start_gather(b)

    def body(i, carry):
        for b in range(NB):
            k = i * NB + b
            _wait_gather(b)
            _start_scatter(b)
            @pl.when(k + NB < NCHUNK)
            def _():
                _wait_scatter(b)
                _start_idx(k + NB, b)
                _wait_idx(k + NB, b)
                _start_gather(b)
        return carry

    lax.fori_loop(0, NCHUNK // NB, body, 0)
    for b in range(LEFT):
        _wait_gather(b)
        _start_scatter(b)
    for b in range(NB):
        _wait_scatter(b)

    tbase = ebase + NCHUNK * CH
    pltpu.sync_copy(src_hbm.at[pl.ds(tbase, TAIL)], sbuf_t)
    pltpu.sync_copy(dst_hbm.at[pl.ds(tbase, TAIL)], dbuf_t)
    trows = rows[0].at[pl.ds(0, TAIL), :]
    pltpu.async_copy(hp_hbm.at[sbuf_t], trows, gsem[0]).wait()
    pltpu.sync_copy(trows, acc.at[dbuf_t], add=True)

    plsc.subcore_barrier()
    for r in range(RCPT):
        chunk = s + r * NS
        @pl.when(chunk < NRCH)
        def _():
            pltpu.sync_copy(
                acc.at[pl.ds(chunk * RZ, RZ), :],
                out_hbm.at[pl.ds(c * N + chunk * RZ, RZ), :],
            )


@functools.cache
def _agg1_kernel():
    return pl.kernel(
        _agg1_body,
        out_type=jax.ShapeDtypeStruct((NW * N,), jnp.float32),
        mesh=plsc.VectorSubcoreMesh(core_axis_name="c", subcore_axis_name="s"),
        compiler_params=pltpu.CompilerParams(needs_layout_passes=False),
        scratch_types=[
            pltpu.VMEM((N,), jnp.float32),
            pltpu.VMEM((N,), jnp.float32),
            pltpu.VMEM((EPW,), jnp.int32),
            pltpu.VMEM((EPW,), jnp.int32),
        ],
    )


def _agg1_body(z_hbm, src_hbm, dst_hbm, out_hbm, zp, acc, sbuf, dbuf):
    c = lax.axis_index("c")
    s = lax.axis_index("s")
    wid = s * NC + c

    pltpu.sync_copy(z_hbm, zp)
    pltpu.sync_copy(src_hbm.at[pl.ds(wid * EPW, EPW)], sbuf)
    pltpu.sync_copy(dst_hbm.at[pl.ds(wid * EPW, EPW)], dbuf)

    def zero_body(i, carry):
        acc[pl.ds(i * 16, 16)] = _zero_vec16()
        return carry

    lax.fori_loop(0, N // 16, zero_body, 0)

    def body(j, carry):
        s16 = sbuf[pl.ds(j * 16, 16)]
        d16 = dbuf[pl.ds(j * 16, 16)]
        vals = plsc.load_gather(zp, [s16])
        plsc.addupdate_scatter(acc, [d16], vals)
        return carry

    lax.fori_loop(0, EPW // 16, body, 0)
    pltpu.sync_copy(acc, out_hbm.at[pl.ds(wid * N, N)])


_RB = 2000


def _prep_body(dp_ref, or_ref, oc_ref):
    deg = jnp.sum(dp_ref[...], axis=0, keepdims=True) + 1.0
    dinv = lax.rsqrt(deg)
    or_ref[...] = dinv
    oc_ref[...] = dinv.T


def _prep(degparts):
    return pl.pallas_call(
        _prep_body,
        out_shape=[
            jax.ShapeDtypeStruct((1, N), jnp.float32),
            jax.ShapeDtypeStruct((N, 1), jnp.float32),
        ],
    )(degparts)


def _mm_body(x_ref, w_ref, dinv_ref, o_ref):
    p = jnp.dot(x_ref[...], w_ref[...], preferred_element_type=jnp.float32)
    o_ref[...] = p * dinv_ref[...]


def _mm(x, W, dinv_col):
    kd = W.shape[0]
    od = W.shape[1]
    return pl.pallas_call(
        _mm_body,
        out_shape=jax.ShapeDtypeStruct((N, od), jnp.float32),
        grid=(N // _RB,),
        in_specs=[
            pl.BlockSpec((_RB, kd), lambda i: (i, 0)),
            pl.BlockSpec((kd, od), lambda i: (0, 0)),
            pl.BlockSpec((_RB, 1), lambda i: (i, 0)),
        ],
        out_specs=pl.BlockSpec((_RB, od), lambda i: (i, 0)),
    )(x, W, dinv_col)


def _combmm_body(p0_ref, p1_ref, pp_ref, dinv_ref, b_ref, w_ref, o_ref):
    t = dinv_ref[...] * (p0_ref[...] + p1_ref[...] + pp_ref[...]) + b_ref[...]
    t = jnp.maximum(t, 0.0)
    p = jnp.dot(t, w_ref[...], preferred_element_type=jnp.float32)
    o_ref[...] = p * dinv_ref[...]


def _combmm(p0, p1, pp, dinv_col, bias_row, Wn):
    od = Wn.shape[1]
    return pl.pallas_call(
        _combmm_body,
        out_shape=jax.ShapeDtypeStruct((N, od), jnp.float32),
        grid=(N // _RB,),
        in_specs=[
            pl.BlockSpec((_RB, D), lambda i: (i, 0)),
            pl.BlockSpec((_RB, D), lambda i: (i, 0)),
            pl.BlockSpec((_RB, D), lambda i: (i, 0)),
            pl.BlockSpec((_RB, 1), lambda i: (i, 0)),
            pl.BlockSpec((1, D), lambda i: (0, 0)),
            pl.BlockSpec((D, od), lambda i: (0, 0)),
        ],
        out_specs=pl.BlockSpec((_RB, od), lambda i: (i, 0)),
    )(p0, p1, pp, dinv_col, bias_row, Wn)


def _comb3_body(parts_ref, z_ref, dinv_ref, b_ref, o_ref):
    agg = jnp.sum(parts_ref[...], axis=0, keepdims=True)
    o_ref[...] = dinv_ref[...] * (agg + z_ref[...]) + b_ref[0, 0]


def _comb3(parts3, z_row, dinv_row, b3):
    return pl.pallas_call(
        _comb3_body,
        out_shape=jax.ShapeDtypeStruct((1, N), jnp.float32),
    )(parts3, z_row, dinv_row, b3)


@jax.jit
def kernel(x, edge_index, W1, b1, W2, b2, W3, b3):
    src = edge_index[0].astype(jnp.int32)
    dst = edge_index[1].astype(jnp.int32)

    degparts = _deg_kernel()(dst).reshape(NW, N)
    dinv_row, dinv_col = _prep(degparts)

    pp1 = _mm(x, W1, dinv_col)
    parts = _agg_kernel()(pp1, src, dst)
    pp2 = _combmm(parts[:N], parts[N:], pp1, dinv_col, b1.reshape(1, D), W2)
    parts = _agg_kernel()(pp2, src, dst)
    z = _combmm(parts[:N], parts[N:], pp2, dinv_col, b2.reshape(1, D), W3)

    parts3 = _agg1_kernel()(z.reshape(N), src, dst).reshape(NW, N)
    out_row = _comb3(parts3, z.reshape(1, N), dinv_row, b3.reshape(1, 1))
    return out_row.reshape(N, 1)


# speedup vs baseline: 1.1949x; 1.1949x over previous
import functools

import jax
import jax.numpy as jnp
from jax import lax
from jax.experimental import pallas as pl
from jax.experimental.pallas import tpu as pltpu
from jax.experimental.pallas import tpu_sc as plsc

N = 10000
E = 320000
D = 128

NC = 2
NS = 16
NW = NC * NS
EPW = E // NW
CH = 128
NCHUNK = EPW // CH
TAIL = EPW - NCHUNK * CH
NB = 3
NI = 2 * NB
LEFT = NCHUNK % NB
RZ = 80

def _zero_vec16():
    return jnp.zeros((16,), jnp.float32)


@functools.cache
def _deg_kernel():
    return pl.kernel(
        _deg_body,
        out_type=jax.ShapeDtypeStruct((NW * N,), jnp.float32),
        mesh=plsc.VectorSubcoreMesh(core_axis_name="c", subcore_axis_name="s"),
        compiler_params=pltpu.CompilerParams(needs_layout_passes=False),
        scratch_types=[
            pltpu.VMEM((N,), jnp.float32),
            pltpu.VMEM((EPW,), jnp.int32),
        ],
    )


def _deg_body(dst_hbm, out_hbm, hist, dbuf):
    c = lax.axis_index("c")
    s = lax.axis_index("s")
    wid = s * NC + c

    pltpu.sync_copy(dst_hbm.at[pl.ds(wid * EPW, EPW)], dbuf)

    def zero_body(i, carry):
        hist[pl.ds(i * 16, 16)] = _zero_vec16()
        return carry

    lax.fori_loop(0, N // 16, zero_body, 0)

    ones = jnp.ones((16,), jnp.float32)

    def body(j, carry):
        d16 = dbuf[pl.ds(j * 16, 16)]
        plsc.addupdate_scatter(hist, [d16], ones)
        return carry

    lax.fori_loop(0, EPW // 16, body, 0)
    pltpu.sync_copy(hist, out_hbm.at[pl.ds(wid * N, N)])


@functools.cache
def _agg_kernel():
    return pl.kernel(
        _agg_body,
        out_type=jax.ShapeDtypeStruct((NC * N, D), jnp.float32),
        mesh=plsc.VectorSubcoreMesh(core_axis_name="c", subcore_axis_name="s"),
        compiler_params=pltpu.CompilerParams(needs_layout_passes=False),
        scratch_types=(
            [pltpu.VMEM_SHARED((N, D), jnp.float32)]
            + [pltpu.VMEM((CH, D), jnp.float32)] * NB
            + [pltpu.VMEM((CH,), jnp.int32)] * NI
            + [pltpu.VMEM((CH,), jnp.int32)] * NI
            + [pltpu.VMEM((TAIL,), jnp.int32)] * 2
            + [pltpu.SemaphoreType.DMA] * (2 * NB + NI)
        ),
    )


NRCH = N // RZ
RCPT = (NRCH + NS - 1) // NS


def _agg_body(hp_hbm, src_hbm, dst_hbm, out_hbm, acc, *bufs):
    o = 0
    rows = bufs[o:o + NB]; o += NB
    sbuf = bufs[o:o + NI]; o += NI
    dbuf = bufs[o:o + NI]; o += NI
    sbuf_t, dbuf_t = bufs[o:o + 2]; o += 2
    gsem = bufs[o:o + NB]; o += NB
    ssem = bufs[o:o + NB]; o += NB
    isem = bufs[o:o + NI]; o += NI
    c = lax.axis_index("c")
    s = lax.axis_index("s")
    wid = s * NC + c
    ebase = wid * EPW

    def zrow(i, carry):
        for j in range(D // 16):
            rows[0][i, pl.ds(j * 16, 16)] = _zero_vec16()
        return carry

    lax.fori_loop(0, RZ, zrow, 0)
    for r in range(RCPT):
        chunk = s + r * NS
        @pl.when(chunk < NRCH)
        def _():
            pltpu.sync_copy(rows[0].at[pl.ds(0, RZ), :], acc.at[pl.ds(chunk * RZ, RZ), :])
    plsc.subcore_barrier()

    def _start_idx(k, j):
        pltpu.async_copy(src_hbm.at[pl.ds(ebase + k * CH, CH)], sbuf[j], isem[j])
        pltpu.async_copy(dst_hbm.at[pl.ds(ebase + k * CH, CH)], dbuf[j], isem[j])

    def _wait_idx(k, j):
        pltpu.make_async_copy(src_hbm.at[pl.ds(ebase + k * CH, CH)], sbuf[j], isem[j]).wait()
        pltpu.make_async_copy(dst_hbm.at[pl.ds(ebase + k * CH, CH)], dbuf[j], isem[j]).wait()

    def _start_gather(b, j):
        pltpu.async_copy(hp_hbm.at[sbuf[j]], rows[b], gsem[b])

    def _wait_gather(b, j):
        pltpu.make_async_copy(hp_hbm.at[sbuf[j]], rows[b], gsem[b]).wait()

    def _start_scatter(b, j):
        pltpu.async_copy(rows[b], acc.at[dbuf[j]], ssem[b], add=True)

    def _wait_scatter(b, j):
        pltpu.make_async_copy(rows[b], acc.at[dbuf[j]], ssem[b]).wait()

    for b in range(NB):
        _start_idx(b, b)
        _wait_idx(b, b)
        _start_gather(b, b)

    def body(i, carry):
        for h in range(2):
            for b in range(NB):
                k = i * NI + h * NB + b
                j = h * NB + b
                _wait_gather(b, j)
                _start_scatter(b, j)
                jn = (j + NB) % NI
                @pl.when(k + NB < NCHUNK)
                def _():
                    _start_idx(k + NB, jn)
                    _wait_scatter(b, j)
                    _wait_idx(k + NB, jn)
                    _start_gather(b, jn)
        return carry

    lax.fori_loop(0, NCHUNK // NI, body, 0)
    done = (NCHUNK // NI) * NI
    for k in range(done, NCHUNK):
        b = k % NB
        j = k % NI
        _wait_gather(b, j)
        _start_scatter(b, j)
    for k in range(NCHUNK - NB, NCHUNK):
        _wait_scatter(k % NB, k % NI)

    tbase = ebase + NCHUNK * CH
    pltpu.sync_copy(src_hbm.at[pl.ds(tbase, TAIL)], sbuf_t)
    pltpu.sync_copy(dst_hbm.at[pl.ds(tbase, TAIL)], dbuf_t)
    trows = rows[0].at[pl.ds(0, TAIL), :]
    pltpu.async_copy(hp_hbm.at[sbuf_t], trows, gsem[0]).wait()
    pltpu.sync_copy(trows, acc.at[dbuf_t], add=True)

    plsc.subcore_barrier()
    for r in range(RCPT):
        chunk = s + r * NS
        @pl.when(chunk < NRCH)
        def _():
            pltpu.sync_copy(
                acc.at[pl.ds(chunk * RZ, RZ), :],
                out_hbm.at[pl.ds(c * N + chunk * RZ, RZ), :],
            )


@functools.cache
def _agg1_kernel():
    return pl.kernel(
        _agg1_body,
        out_type=jax.ShapeDtypeStruct((NW * N,), jnp.float32),
        mesh=plsc.VectorSubcoreMesh(core_axis_name="c", subcore_axis_name="s"),
        compiler_params=pltpu.CompilerParams(needs_layout_passes=False),
        scratch_types=[
            pltpu.VMEM((N,), jnp.float32),
            pltpu.VMEM((N,), jnp.float32),
            pltpu.VMEM((EPW,), jnp.int32),
            pltpu.VMEM((EPW,), jnp.int32),
        ],
    )


def _agg1_body(z_hbm, src_hbm, dst_hbm, out_hbm, zp, acc, sbuf, dbuf):
    c = lax.axis_index("c")
    s = lax.axis_index("s")
    wid = s * NC + c

    pltpu.sync_copy(z_hbm, zp)
    pltpu.sync_copy(src_hbm.at[pl.ds(wid * EPW, EPW)], sbuf)
    pltpu.sync_copy(dst_hbm.at[pl.ds(wid * EPW, EPW)], dbuf)

    def zero_body(i, carry):
        acc[pl.ds(i * 16, 16)] = _zero_vec16()
        return carry

    lax.fori_loop(0, N // 16, zero_body, 0)

    def body(j, carry):
        s16 = sbuf[pl.ds(j * 16, 16)]
        d16 = dbuf[pl.ds(j * 16, 16)]
        vals = plsc.load_gather(zp, [s16])
        plsc.addupdate_scatter(acc, [d16], vals)
        return carry

    lax.fori_loop(0, EPW // 16, body, 0)
    pltpu.sync_copy(acc, out_hbm.at[pl.ds(wid * N, N)])


_RB = 2000


def _prep_body(dp_ref, or_ref, oc_ref):
    deg = jnp.sum(dp_ref[...], axis=0, keepdims=True) + 1.0
    dinv = lax.rsqrt(deg)
    or_ref[...] = dinv
    oc_ref[...] = dinv.T


def _prep(degparts):
    return pl.pallas_call(
        _prep_body,
        out_shape=[
            jax.ShapeDtypeStruct((1, N), jnp.float32),
            jax.ShapeDtypeStruct((N, 1), jnp.float32),
        ],
    )(degparts)


def _mm_body(x_ref, w_ref, dinv_ref, o_ref):
    p = jnp.dot(x_ref[...], w_ref[...], preferred_element_type=jnp.float32)
    o_ref[...] = p * dinv_ref[...]


def _mm(x, W, dinv_col):
    kd = W.shape[0]
    od = W.shape[1]
    return pl.pallas_call(
        _mm_body,
        out_shape=jax.ShapeDtypeStruct((N, od), jnp.float32),
        grid=(N // _RB,),
        in_specs=[
            pl.BlockSpec((_RB, kd), lambda i: (i, 0)),
            pl.BlockSpec((kd, od), lambda i: (0, 0)),
            pl.BlockSpec((_RB, 1), lambda i: (i, 0)),
        ],
        out_specs=pl.BlockSpec((_RB, od), lambda i: (i, 0)),
    )(x, W, dinv_col)


def _combmm_body(p0_ref, p1_ref, pp_ref, dinv_ref, b_ref, w_ref, o_ref):
    t = dinv_ref[...] * (p0_ref[...] + p1_ref[...] + pp_ref[...]) + b_ref[...]
    t = jnp.maximum(t, 0.0)
    p = jnp.dot(t, w_ref[...], preferred_element_type=jnp.float32)
    o_ref[...] = p * dinv_ref[...]


def _combmm(p0, p1, pp, dinv_col, bias_row, Wn):
    od = Wn.shape[1]
    return pl.pallas_call(
        _combmm_body,
        out_shape=jax.ShapeDtypeStruct((N, od), jnp.float32),
        grid=(N // _RB,),
        in_specs=[
            pl.BlockSpec((_RB, D), lambda i: (i, 0)),
            pl.BlockSpec((_RB, D), lambda i: (i, 0)),
            pl.BlockSpec((_RB, D), lambda i: (i, 0)),
            pl.BlockSpec((_RB, 1), lambda i: (i, 0)),
            pl.BlockSpec((1, D), lambda i: (0, 0)),
            pl.BlockSpec((D, od), lambda i: (0, 0)),
        ],
        out_specs=pl.BlockSpec((_RB, od), lambda i: (i, 0)),
    )(p0, p1, pp, dinv_col, bias_row, Wn)


def _comb3_body(parts_ref, z_ref, dinv_ref, b_ref, o_ref):
    agg = jnp.sum(parts_ref[...], axis=0, keepdims=True)
    o_ref[...] = dinv_ref[...] * (agg + z_ref[...]) + b_ref[0, 0]


def _comb3(parts3, z_row, dinv_row, b3):
    return pl.pallas_call(
        _comb3_body,
        out_shape=jax.ShapeDtypeStruct((1, N), jnp.float32),
    )(parts3, z_row, dinv_row, b3)


@jax.jit
def kernel(x, edge_index, W1, b1, W2, b2, W3, b3):
    src = edge_index[0].astype(jnp.int32)
    dst = edge_index[1].astype(jnp.int32)

    degparts = _deg_kernel()(dst).reshape(NW, N)
    dinv_row, dinv_col = _prep(degparts)

    pp1 = _mm(x, W1, dinv_col)
    parts = _agg_kernel()(pp1, src, dst)
    pp2 = _combmm(parts[:N], parts[N:], pp1, dinv_col, b1.reshape(1, D), W2)
    parts = _agg_kernel()(pp2, src, dst)
    z = _combmm(parts[:N], parts[N:], pp2, dinv_col, b2.reshape(1, D), W3)

    parts3 = _agg1_kernel()(z.reshape(N), src, dst).reshape(NW, N)
    out_row = _comb3(parts3, z.reshape(1, N), dinv_row, b3.reshape(1, 1))
    return out_row.reshape(N, 1)


# speedup vs baseline: 1.3496x; 1.1295x over previous
import functools

import jax
import jax.numpy as jnp
from jax import lax
from jax.experimental import pallas as pl
from jax.experimental.pallas import tpu as pltpu
from jax.experimental.pallas import tpu_sc as plsc

N = 10000
E = 320000
D = 128

NC = 2
NS = 16
NW = NC * NS
EPW = E // NW
CH = 128
NCHUNK = EPW // CH
TAIL = EPW - NCHUNK * CH
NB = 3
NI = 2 * NB
LEFT = NCHUNK % NB
RZ = 80

def _zero_vec16():
    return jnp.zeros((16,), jnp.float32)


@functools.cache
def _deg_kernel():
    return pl.kernel(
        _deg_body,
        out_type=jax.ShapeDtypeStruct((NW, 1, N), jnp.float32),
        mesh=plsc.VectorSubcoreMesh(core_axis_name="c", subcore_axis_name="s"),
        compiler_params=pltpu.CompilerParams(needs_layout_passes=False),
        scratch_types=[
            pltpu.VMEM((1, N), jnp.float32),
            pltpu.VMEM((EPW,), jnp.int32),
        ],
    )


def _deg_body(ei_hbm, out_hbm, hist, dbuf):
    c = lax.axis_index("c")
    s = lax.axis_index("s")
    wid = s * NC + c

    pltpu.sync_copy(ei_hbm.at[pl.ds(E + wid * EPW, EPW)], dbuf)

    def zero_body(i, carry):
        hist[0, pl.ds(i * 16, 16)] = _zero_vec16()
        return carry

    lax.fori_loop(0, N // 16, zero_body, 0)

    ones = jnp.ones((16,), jnp.float32)
    zi16 = jnp.zeros((16,), jnp.int32)

    def body(j, carry):
        d16 = dbuf[pl.ds(j * 16, 16)]
        plsc.addupdate_scatter(hist, [zi16, d16], ones)
        return carry

    lax.fori_loop(0, EPW // 16, body, 0)
    pltpu.sync_copy(hist, out_hbm.at[wid])


@functools.cache
def _agg_kernel():
    return pl.kernel(
        _agg_body,
        out_type=jax.ShapeDtypeStruct((NC * N, D), jnp.float32),
        mesh=plsc.VectorSubcoreMesh(core_axis_name="c", subcore_axis_name="s"),
        compiler_params=pltpu.CompilerParams(needs_layout_passes=False),
        scratch_types=(
            [pltpu.VMEM_SHARED((N, D), jnp.float32)]
            + [pltpu.VMEM((CH, D), jnp.float32)] * NB
            + [pltpu.VMEM((CH,), jnp.int32)] * NI
            + [pltpu.VMEM((CH,), jnp.int32)] * NI
            + [pltpu.VMEM((TAIL,), jnp.int32)] * 2
            + [pltpu.SemaphoreType.DMA] * (2 * NB + NI)
        ),
    )


NRCH = N // RZ
RCPT = (NRCH + NS - 1) // NS


def _agg_body(hp_hbm, ei_hbm, out_hbm, acc, *bufs):
    o = 0
    rows = bufs[o:o + NB]; o += NB
    sbuf = bufs[o:o + NI]; o += NI
    dbuf = bufs[o:o + NI]; o += NI
    sbuf_t, dbuf_t = bufs[o:o + 2]; o += 2
    gsem = bufs[o:o + NB]; o += NB
    ssem = bufs[o:o + NB]; o += NB
    isem = bufs[o:o + NI]; o += NI
    c = lax.axis_index("c")
    s = lax.axis_index("s")
    wid = s * NC + c
    ebase = wid * EPW

    def zrow(i, carry):
        for j in range(D // 16):
            rows[0][i, pl.ds(j * 16, 16)] = _zero_vec16()
        return carry

    lax.fori_loop(0, RZ, zrow, 0)
    for r in range(RCPT):
        chunk = s + r * NS
        @pl.when(chunk < NRCH)
        def _():
            pltpu.sync_copy(rows[0].at[pl.ds(0, RZ), :], acc.at[pl.ds(chunk * RZ, RZ), :])
    plsc.subcore_barrier()

    def _start_idx(k, j):
        pltpu.async_copy(ei_hbm.at[pl.ds(ebase + k * CH, CH)], sbuf[j], isem[j])
        pltpu.async_copy(ei_hbm.at[pl.ds(E + ebase + k * CH, CH)], dbuf[j], isem[j])

    def _wait_idx(k, j):
        pltpu.make_async_copy(ei_hbm.at[pl.ds(ebase + k * CH, CH)], sbuf[j], isem[j]).wait()
        pltpu.make_async_copy(ei_hbm.at[pl.ds(E + ebase + k * CH, CH)], dbuf[j], isem[j]).wait()

    def _start_gather(b, j):
        pltpu.async_copy(hp_hbm.at[sbuf[j]], rows[b], gsem[b])

    def _wait_gather(b, j):
        pltpu.make_async_copy(hp_hbm.at[sbuf[j]], rows[b], gsem[b]).wait()

    def _start_scatter(b, j):
        pltpu.async_copy(rows[b], acc.at[dbuf[j]], ssem[b], add=True)

    def _wait_scatter(b, j):
        pltpu.make_async_copy(rows[b], acc.at[dbuf[j]], ssem[b]).wait()

    for b in range(NB):
        _start_idx(b, b)
        _wait_idx(b, b)
        _start_gather(b, b)

    def body(i, carry):
        for h in range(2):
            for b in range(NB):
                k = i * NI + h * NB + b
                j = h * NB + b
                _wait_gather(b, j)
                _start_scatter(b, j)
                jn = (j + NB) % NI
                @pl.when(k + NB < NCHUNK)
                def _():
                    _start_idx(k + NB, jn)
                    _wait_scatter(b, j)
                    _wait_idx(k + NB, jn)
                    _start_gather(b, jn)
        return carry

    lax.fori_loop(0, NCHUNK // NI, body, 0)
    done = (NCHUNK // NI) * NI
    for k in range(done, NCHUNK):
        b = k % NB
        j = k % NI
        _wait_gather(b, j)
        _start_scatter(b, j)
    for k in range(NCHUNK - NB, NCHUNK):
        _wait_scatter(k % NB, k % NI)

    tbase = ebase + NCHUNK * CH
    pltpu.sync_copy(ei_hbm.at[pl.ds(tbase, TAIL)], sbuf_t)
    pltpu.sync_copy(ei_hbm.at[pl.ds(E + tbase, TAIL)], dbuf_t)
    trows = rows[0].at[pl.ds(0, TAIL), :]
    pltpu.async_copy(hp_hbm.at[sbuf_t], trows, gsem[0]).wait()
    pltpu.sync_copy(trows, acc.at[dbuf_t], add=True)

    plsc.subcore_barrier()
    for r in range(RCPT):
        chunk = s + r * NS
        @pl.when(chunk < NRCH)
        def _():
            pltpu.sync_copy(
                acc.at[pl.ds(chunk * RZ, RZ), :],
                out_hbm.at[pl.ds(c * N + chunk * RZ, RZ), :],
            )


@functools.cache
def _agg1_kernel():
    return pl.kernel(
        _agg1_body,
        out_type=jax.ShapeDtypeStruct((NW, 1, N), jnp.float32),
        mesh=plsc.VectorSubcoreMesh(core_axis_name="c", subcore_axis_name="s"),
        compiler_params=pltpu.CompilerParams(needs_layout_passes=False),
        scratch_types=[
            pltpu.VMEM((N,), jnp.float32),
            pltpu.VMEM((1, N), jnp.float32),
            pltpu.VMEM((EPW,), jnp.int32),
            pltpu.VMEM((EPW,), jnp.int32),
        ],
    )


def _agg1_body(z_hbm, ei_hbm, out_hbm, zp, acc, sbuf, dbuf):
    c = lax.axis_index("c")
    s = lax.axis_index("s")
    wid = s * NC + c

    pltpu.sync_copy(z_hbm, zp)
    pltpu.sync_copy(ei_hbm.at[pl.ds(wid * EPW, EPW)], sbuf)
    pltpu.sync_copy(ei_hbm.at[pl.ds(E + wid * EPW, EPW)], dbuf)

    def zero_body(i, carry):
        acc[0, pl.ds(i * 16, 16)] = _zero_vec16()
        return carry

    lax.fori_loop(0, N // 16, zero_body, 0)

    zi16 = jnp.zeros((16,), jnp.int32)

    def body(j, carry):
        s16 = sbuf[pl.ds(j * 16, 16)]
        d16 = dbuf[pl.ds(j * 16, 16)]
        vals = plsc.load_gather(zp, [s16])
        plsc.addupdate_scatter(acc, [zi16, d16], vals)
        return carry

    lax.fori_loop(0, EPW // 16, body, 0)
    pltpu.sync_copy(acc, out_hbm.at[wid])


_RB = 2000


def _prep_body(dp_ref, or_ref, oc_ref):
    deg = jnp.sum(dp_ref[...], axis=0) + 1.0
    dinv = lax.rsqrt(deg)
    or_ref[...] = dinv
    oc_ref[...] = dinv.T


def _prep(degparts):
    return pl.pallas_call(
        _prep_body,
        out_shape=[
            jax.ShapeDtypeStruct((1, N), jnp.float32),
            jax.ShapeDtypeStruct((N, 1), jnp.float32),
        ],
    )(degparts)


def _mm_body(x_ref, w_ref, dinv_ref, o_ref):
    p = jnp.dot(x_ref[...], w_ref[...], preferred_element_type=jnp.float32)
    o_ref[...] = p * dinv_ref[...]


def _mm(x, W, dinv_col):
    kd = W.shape[0]
    od = W.shape[1]
    return pl.pallas_call(
        _mm_body,
        out_shape=jax.ShapeDtypeStruct((N, od), jnp.float32),
        grid=(N // _RB,),
        in_specs=[
            pl.BlockSpec((_RB, kd), lambda i: (i, 0)),
            pl.BlockSpec((kd, od), lambda i: (0, 0)),
            pl.BlockSpec((_RB, 1), lambda i: (i, 0)),
        ],
        out_specs=pl.BlockSpec((_RB, od), lambda i: (i, 0)),
    )(x, W, dinv_col)


def _combmm_body(p0_ref, p1_ref, pp_ref, dinv_ref, b_ref, w_ref, o_ref):
    t = dinv_ref[...] * (p0_ref[...] + p1_ref[...] + pp_ref[...]) + b_ref[...]
    t = jnp.maximum(t, 0.0)
    p = jnp.dot(t, w_ref[...], preferred_element_type=jnp.float32)
    o_ref[...] = p * dinv_ref[...]


def _combmm(parts, pp, dinv_col, bias_row, Wn):
    hb = N // _RB
    return pl.pallas_call(
        _combmm_body,
        out_shape=jax.ShapeDtypeStruct((N, D), jnp.float32),
        grid=(N // _RB,),
        in_specs=[
            pl.BlockSpec((_RB, D), lambda i: (i, 0)),
            pl.BlockSpec((_RB, D), lambda i: (i + hb, 0)),
            pl.BlockSpec((_RB, D), lambda i: (i, 0)),
            pl.BlockSpec((_RB, 1), lambda i: (i, 0)),
            pl.BlockSpec((1, D), lambda i: (0, 0)),
            pl.BlockSpec((D, D), lambda i: (0, 0)),
        ],
        out_specs=pl.BlockSpec((_RB, D), lambda i: (i, 0)),
    )(parts, parts, pp, dinv_col, bias_row, Wn)


def _combz_body(parts_ref, pp_ref, dinv_ref, b_ref, w_ref, o_ref):
    agg = parts_ref[pl.ds(0, N), :] + parts_ref[pl.ds(N, N), :]
    t = dinv_ref[...] * (agg + pp_ref[...]) + b_ref[...]
    t = jnp.maximum(t, 0.0)
    z = jnp.sum(t * w_ref[...], axis=1)
    o_ref[...] = z * dinv_ref[...][:, 0]


def _combz(parts, pp, dinv_col, bias_row, w_row):
    return pl.pallas_call(
        _combz_body,
        out_shape=jax.ShapeDtypeStruct((N,), jnp.float32),
    )(parts, pp, dinv_col, bias_row, w_row)


def _comb3_body(parts_ref, z_ref, dinv_ref, b_ref, o_ref):
    agg = jnp.sum(parts_ref[...], axis=0)
    z = z_ref[...].reshape(1, N)
    o_ref[...] = dinv_ref[...] * (agg + z) + b_ref[0, 0]


def _comb3(parts3, z_flat, dinv_row, b3):
    return pl.pallas_call(
        _comb3_body,
        out_shape=jax.ShapeDtypeStruct((1, N), jnp.float32),
    )(parts3, z_flat, dinv_row, b3)


@jax.jit
def kernel(x, edge_index, W1, b1, W2, b2, W3, b3):
    ei = edge_index.astype(jnp.int32).reshape(2 * E)

    degparts = _deg_kernel()(ei)
    dinv_row, dinv_col = _prep(degparts)

    pp1 = _mm(x, W1, dinv_col)
    parts = _agg_kernel()(pp1, ei)
    pp2 = _combmm(parts, pp1, dinv_col, b1.reshape(1, D), W2)
    parts = _agg_kernel()(pp2, ei)
    z = _combz(parts, pp2, dinv_col, b2.reshape(1, D), W3.reshape(1, D))

    parts3 = _agg1_kernel()(z, ei)
    out_row = _comb3(parts3, z, dinv_row, b3.reshape(1, 1))
    return out_row.reshape(N, 1)


# speedup vs baseline: 1.3523x; 1.0020x over previous
import functools

import jax
import jax.numpy as jnp
from jax import lax
from jax.experimental import pallas as pl
from jax.experimental.pallas import tpu as pltpu
from jax.experimental.pallas import tpu_sc as plsc

N = 10000
E = 320000
D = 128

NC = 2
NS = 16
NW = NC * NS
EPW = E // NW
CH = 128
NCHUNK = EPW // CH
TAIL = EPW - NCHUNK * CH
NB = 3
NI = 2 * NB
LEFT = NCHUNK % NB
RZ = 80

def _zero_vec16():
    return jnp.zeros((16,), jnp.float32)


@functools.cache
def _deg_kernel():
    return pl.kernel(
        _deg_body,
        out_type=jax.ShapeDtypeStruct((NW, 1, N), jnp.float32),
        mesh=plsc.VectorSubcoreMesh(core_axis_name="c", subcore_axis_name="s"),
        compiler_params=pltpu.CompilerParams(needs_layout_passes=False),
        scratch_types=[
            pltpu.VMEM((1, N), jnp.float32),
            pltpu.VMEM((EPW,), jnp.int32),
        ],
    )


def _deg_body(ei_hbm, out_hbm, hist, dbuf):
    c = lax.axis_index("c")
    s = lax.axis_index("s")
    wid = s * NC + c

    pltpu.sync_copy(ei_hbm.at[pl.ds(E + wid * EPW, EPW)], dbuf)

    def zero_body(i, carry):
        hist[0, pl.ds(i * 16, 16)] = _zero_vec16()
        return carry

    lax.fori_loop(0, N // 16, zero_body, 0)

    ones = jnp.ones((16,), jnp.float32)
    zi16 = jnp.zeros((16,), jnp.int32)

    def body(j, carry):
        d16 = dbuf[pl.ds(j * 16, 16)]
        plsc.addupdate_scatter(hist, [zi16, d16], ones)
        return carry

    lax.fori_loop(0, EPW // 16, body, 0)
    pltpu.sync_copy(hist, out_hbm.at[wid])


@functools.cache
def _agg_kernel():
    return pl.kernel(
        _agg_body,
        out_type=jax.ShapeDtypeStruct((NC * N, D), jnp.float32),
        mesh=plsc.VectorSubcoreMesh(core_axis_name="c", subcore_axis_name="s"),
        compiler_params=pltpu.CompilerParams(needs_layout_passes=False),
        scratch_types=(
            [pltpu.VMEM_SHARED((N, D), jnp.float32)]
            + [pltpu.VMEM((CH, D), jnp.float32)] * NB
            + [pltpu.VMEM((CH,), jnp.int32)] * NI
            + [pltpu.VMEM((CH,), jnp.int32)] * NI
            + [pltpu.VMEM((TAIL,), jnp.int32)] * 2
            + [pltpu.SemaphoreType.DMA] * (2 * NB + NI)
        ),
    )


NRCH = N // RZ
RCPT = (NRCH + NS - 1) // NS


def _agg_body(hp_hbm, ei_hbm, out_hbm, acc, *bufs):
    o = 0
    rows = bufs[o:o + NB]; o += NB
    sbuf = bufs[o:o + NI]; o += NI
    dbuf = bufs[o:o + NI]; o += NI
    sbuf_t, dbuf_t = bufs[o:o + 2]; o += 2
    gsem = bufs[o:o + NB]; o += NB
    ssem = bufs[o:o + NB]; o += NB
    isem = bufs[o:o + NI]; o += NI
    c = lax.axis_index("c")
    s = lax.axis_index("s")
    wid = s * NC + c
    ebase = wid * EPW

    def zrow(i, carry):
        for j in range(D // 16):
            rows[0][i, pl.ds(j * 16, 16)] = _zero_vec16()
        return carry

    lax.fori_loop(0, RZ, zrow, 0)
    for r in range(RCPT):
        chunk = s + r * NS
        @pl.when(chunk < NRCH)
        def _():
            pltpu.sync_copy(rows[0].at[pl.ds(0, RZ), :], acc.at[pl.ds(chunk * RZ, RZ), :])
    plsc.subcore_barrier()

    def _start_idx(k, j):
        pltpu.async_copy(ei_hbm.at[pl.ds(ebase + k * CH, CH)], sbuf[j], isem[j])
        pltpu.async_copy(ei_hbm.at[pl.ds(E + ebase + k * CH, CH)], dbuf[j], isem[j])

    def _wait_idx(k, j):
        pltpu.make_async_copy(ei_hbm.at[pl.ds(ebase + k * CH, CH)], sbuf[j], isem[j]).wait()
        pltpu.make_async_copy(ei_hbm.at[pl.ds(E + ebase + k * CH, CH)], dbuf[j], isem[j]).wait()

    def _start_gather(b, j):
        pltpu.async_copy(hp_hbm.at[sbuf[j]], rows[b], gsem[b])

    def _wait_gather(b, j):
        pltpu.make_async_copy(hp_hbm.at[sbuf[j]], rows[b], gsem[b]).wait()

    def _start_scatter(b, j):
        pltpu.async_copy(rows[b], acc.at[dbuf[j]], ssem[b], add=True)

    def _wait_scatter(b, j):
        pltpu.make_async_copy(rows[b], acc.at[dbuf[j]], ssem[b]).wait()

    for b in range(NB):
        _start_idx(b, b)
        _wait_idx(b, b)
        _start_gather(b, b)

    def body(i, carry):
        for h in range(2):
            for b in range(NB):
                k = i * NI + h * NB + b
                j = h * NB + b
                _wait_gather(b, j)
                _start_scatter(b, j)
                jn = (j + NB) % NI
                @pl.when(k + NB < NCHUNK)
                def _():
                    _start_idx(k + NB, jn)
                    _wait_scatter(b, j)
                    _wait_idx(k + NB, jn)
                    _start_gather(b, jn)
        return carry

    lax.fori_loop(0, NCHUNK // NI, body, 0)
    done = (NCHUNK // NI) * NI
    for k in range(done, NCHUNK):
        b = k % NB
        j = k % NI
        _wait_gather(b, j)
        _start_scatter(b, j)
    for k in range(NCHUNK - NB, NCHUNK):
        _wait_scatter(k % NB, k % NI)

    tbase = ebase + NCHUNK * CH
    pltpu.sync_copy(ei_hbm.at[pl.ds(tbase, TAIL)], sbuf_t)
    pltpu.sync_copy(ei_hbm.at[pl.ds(E + tbase, TAIL)], dbuf_t)
    trows = rows[0].at[pl.ds(0, TAIL), :]
    pltpu.async_copy(hp_hbm.at[sbuf_t], trows, gsem[0]).wait()
    pltpu.sync_copy(trows, acc.at[dbuf_t], add=True)

    plsc.subcore_barrier()
    for r in range(RCPT):
        chunk = s + r * NS
        @pl.when(chunk < NRCH)
        def _():
            pltpu.sync_copy(
                acc.at[pl.ds(chunk * RZ, RZ), :],
                out_hbm.at[pl.ds(c * N + chunk * RZ, RZ), :],
            )


@functools.cache
def _agg1_kernel():
    return pl.kernel(
        _agg1_body,
        out_type=jax.ShapeDtypeStruct((NW, 1, N), jnp.float32),
        mesh=plsc.VectorSubcoreMesh(core_axis_name="c", subcore_axis_name="s"),
        compiler_params=pltpu.CompilerParams(needs_layout_passes=False),
        scratch_types=[
            pltpu.VMEM((N,), jnp.float32),
            pltpu.VMEM((1, N), jnp.float32),
            pltpu.VMEM((EPW,), jnp.int32),
            pltpu.VMEM((EPW,), jnp.int32),
        ],
    )


def _agg1_body(z_hbm, ei_hbm, out_hbm, zp, acc, sbuf, dbuf):
    c = lax.axis_index("c")
    s = lax.axis_index("s")
    wid = s * NC + c

    pltpu.sync_copy(z_hbm, zp)
    pltpu.sync_copy(ei_hbm.at[pl.ds(wid * EPW, EPW)], sbuf)
    pltpu.sync_copy(ei_hbm.at[pl.ds(E + wid * EPW, EPW)], dbuf)

    def zero_body(i, carry):
        acc[0, pl.ds(i * 16, 16)] = _zero_vec16()
        return carry

    lax.fori_loop(0, N // 16, zero_body, 0)

    zi16 = jnp.zeros((16,), jnp.int32)

    def body(j, carry):
        s16 = sbuf[pl.ds(j * 16, 16)]
        d16 = dbuf[pl.ds(j * 16, 16)]
        vals = plsc.load_gather(zp, [s16])
        plsc.addupdate_scatter(acc, [zi16, d16], vals)
        return carry

    lax.fori_loop(0, EPW // 16, body, 0)
    pltpu.sync_copy(acc, out_hbm.at[wid])


_RB = 2000


def _prep_body(dp_ref, or_ref, oc_ref):
    deg = jnp.sum(dp_ref[...], axis=0) + 1.0
    dinv = lax.rsqrt(deg)
    or_ref[...] = dinv
    oc_ref[...] = dinv.T


def _prep(degparts):
    return pl.pallas_call(
        _prep_body,
        out_shape=[
            jax.ShapeDtypeStruct((1, N), jnp.float32),
            jax.ShapeDtypeStruct((N, 1), jnp.float32),
        ],
    )(degparts)


def _mm_body(x_ref, w_ref, dinv_ref, o_ref):
    p = jnp.dot(x_ref[...], w_ref[...], preferred_element_type=jnp.float32)
    o_ref[...] = p * dinv_ref[...]


def _mm(x, W, dinv_col):
    kd = W.shape[0]
    od = W.shape[1]
    return pl.pallas_call(
        _mm_body,
        out_shape=jax.ShapeDtypeStruct((N, od), jnp.float32),
        grid=(N // _RB,),
        in_specs=[
            pl.BlockSpec((_RB, kd), lambda i: (i, 0)),
            pl.BlockSpec((kd, od), lambda i: (0, 0)),
            pl.BlockSpec((_RB, 1), lambda i: (i, 0)),
        ],
        out_specs=pl.BlockSpec((_RB, od), lambda i: (i, 0)),
    )(x, W, dinv_col)


def _combmm_body(p0_ref, p1_ref, pp_ref, dinv_ref, b_ref, w_ref, o_ref):
    t = dinv_ref[...] * (p0_ref[...] + p1_ref[...] + pp_ref[...]) + b_ref[...]
    t = jnp.maximum(t, 0.0)
    p = jnp.dot(t, w_ref[...], preferred_element_type=jnp.float32)
    o_ref[...] = p * dinv_ref[...]


def _combmm(parts, pp, dinv_col, bias_row, Wn):
    hb = N // _RB
    return pl.pallas_call(
        _combmm_body,
        out_shape=jax.ShapeDtypeStruct((N, D), jnp.float32),
        grid=(N // _RB,),
        in_specs=[
            pl.BlockSpec((_RB, D), lambda i: (i, 0)),
            pl.BlockSpec((_RB, D), lambda i: (i + hb, 0)),
            pl.BlockSpec((_RB, D), lambda i: (i, 0)),
            pl.BlockSpec((_RB, 1), lambda i: (i, 0)),
            pl.BlockSpec((1, D), lambda i: (0, 0)),
            pl.BlockSpec((D, D), lambda i: (0, 0)),
        ],
        out_specs=pl.BlockSpec((_RB, D), lambda i: (i, 0)),
    )(parts, parts, pp, dinv_col, bias_row, Wn)


def _combz_body(parts_ref, pp_ref, dinv_ref, b_ref, w_ref, o_ref):
    agg = parts_ref[pl.ds(0, N), :] + parts_ref[pl.ds(N, N), :]
    t = dinv_ref[...] * (agg + pp_ref[...]) + b_ref[...]
    t = jnp.maximum(t, 0.0)
    z = jnp.dot(t, w_ref[...], preferred_element_type=jnp.float32)
    o_ref[...] = (z * dinv_ref[...])[:, 0]


def _combz(parts, pp, dinv_col, bias_row, w_col):
    return pl.pallas_call(
        _combz_body,
        out_shape=jax.ShapeDtypeStruct((N,), jnp.float32),
    )(parts, pp, dinv_col, bias_row, w_col)


def _comb3_body(parts_ref, z_ref, dinv_ref, b_ref, o_ref):
    agg = jnp.sum(parts_ref[...], axis=0)
    z = z_ref[...].reshape(1, N)
    o_ref[...] = dinv_ref[...] * (agg + z) + b_ref[0, 0]


def _comb3(parts3, z_flat, dinv_row, b3):
    return pl.pallas_call(
        _comb3_body,
        out_shape=jax.ShapeDtypeStruct((1, N), jnp.float32),
    )(parts3, z_flat, dinv_row, b3)


@jax.jit
def kernel(x, edge_index, W1, b1, W2, b2, W3, b3):
    ei = edge_index.astype(jnp.int32).reshape(2 * E)

    degparts = _deg_kernel()(ei)
    dinv_row, dinv_col = _prep(degparts)

    pp1 = _mm(x, W1, dinv_col)
    parts = _agg_kernel()(pp1, ei)
    pp2 = _combmm(parts, pp1, dinv_col, b1.reshape(1, D), W2)
    parts = _agg_kernel()(pp2, ei)
    z = _combz(parts, pp2, dinv_col, b2.reshape(1, D), W3)

    parts3 = _agg1_kernel()(z, ei)
    out_row = _comb3(parts3, z, dinv_row, b3.reshape(1, 1))
    return out_row.reshape(N, 1)
